# initial kernel scaffold (unmeasured)
import jax
import jax.numpy as jnp
from jax import lax
from jax.experimental import pallas as pl
from jax.experimental.pallas import tpu as pltpu

N_DEV = 8
SQ = 256
SKV = 4096
H = 8
DH = 128
D = 1024
KV_BLK = 1024
SCALE = 0.08838834764831843
NEG_BIG = -1e30


def kernel(x, Wq, Wo, K_ext, V_ext):
    def body(x_ref, wq_ref, wo_ref, k_ref, v_ref, out_ref,
             q_buf, o_buf, ml_buf, fin_buf,
             q_send, q_recv, o_send, o_recv, ml_send, ml_recv,
             fin_send, fin_recv):
        me = lax.axis_index("i")
        right = lax.rem(me + 1, N_DEV)
        left = lax.rem(me + N_DEV - 1, N_DEV)

        barrier = pltpu.get_barrier_semaphore()
        pl.semaphore_signal(barrier, inc=1, device_id=(left,),
                            device_id_type=pl.DeviceIdType.MESH)
        pl.semaphore_signal(barrier, inc=1, device_id=(right,),
                            device_id_type=pl.DeviceIdType.MESH)
        pl.semaphore_wait(barrier, 2)

        q0 = jnp.dot(x_ref[0], wq_ref[...],
                     preferred_element_type=jnp.float32)
        q_buf[0] = q0.reshape(SQ, H, DH).transpose(1, 0, 2)

        def flash_update(q, o, m, l):
            def step(j, carry):
                o, m, l = carry
                kj = k_ref[0, pl.ds(j * KV_BLK, KV_BLK)]
                vj = v_ref[0, pl.ds(j * KV_BLK, KV_BLK)]
                s = lax.dot_general(
                    q, kj, (((2,), (2,)), ((0,), (1,))),
                    preferred_element_type=jnp.float32) * SCALE
                mj = jnp.max(s, axis=-1)
                m_new = jnp.maximum(m, mj)
                alpha = jnp.exp(m - m_new)
                p = jnp.exp(s - m_new[:, :, None])
                l_new = l * alpha + jnp.sum(p, axis=-1)
                pv = lax.dot_general(
                    p, vj, (((2,), (0,)), ((0,), (1,))),
                    preferred_element_type=jnp.float32)
                o_new = o * alpha[:, :, None] + pv
                return o_new, m_new, l_new

            return lax.fori_loop(0, SKV // KV_BLK, step, (o, m, l))

        for h in range(N_DEV):
            q = q_buf[h]
            if h == 0:
                o = jnp.zeros((H, SQ, DH), jnp.float32)
                m = jnp.full((H, SQ), NEG_BIG, jnp.float32)
                l = jnp.zeros((H, SQ), jnp.float32)
            else:
                o = o_buf[h]
                m = ml_buf[h, 0]
                l = ml_buf[h, 1]

            o, m, l = flash_update(q, o, m, l)

            if h < N_DEV - 1:
                o_buf[h] = o
                ml_buf[h, 0] = m
                ml_buf[h, 1] = l
                rq = pltpu.make_async_remote_copy(
                    src_ref=q_buf.at[h], dst_ref=q_buf.at[h + 1],
                    send_sem=q_send.at[h], recv_sem=q_recv.at[h + 1],
                    device_id=(right,),
                    device_id_type=pl.DeviceIdType.MESH)
                ro = pltpu.make_async_remote_copy(
                    src_ref=o_buf.at[h], dst_ref=o_buf.at[h + 1],
                    send_sem=o_send.at[h], recv_sem=o_recv.at[h + 1],
                    device_id=(right,),
                    device_id_type=pl.DeviceIdType.MESH)
                rml = pltpu.make_async_remote_copy(
                    src_ref=ml_buf.at[h], dst_ref=ml_buf.at[h + 1],
                    send_sem=ml_send.at[h], recv_sem=ml_recv.at[h + 1],
                    device_id=(right,),
                    device_id_type=pl.DeviceIdType.MESH)
                rq.start()
                ro.start()
                rml.start()
                rq.wait()
                ro.wait()
                rml.wait()
            else:
                o = o / l[:, :, None]
                res = o.transpose(1, 0, 2).reshape(SQ, D)
                fin_buf[...] = jnp.dot(res, wo_ref[...],
                                       preferred_element_type=jnp.float32)
                rfin = pltpu.make_async_remote_copy(
                    src_ref=fin_buf, dst_ref=out_ref.at[0],
                    send_sem=fin_send, recv_sem=fin_recv,
                    device_id=(right,),
                    device_id_type=pl.DeviceIdType.MESH)
                rfin.start()
                rfin.wait()

    return pl.pallas_call(
        body,
        out_shape=jax.ShapeDtypeStruct((1, SQ, D), jnp.float32),
        in_specs=[pl.BlockSpec(memory_space=pltpu.VMEM)] * 5,
        out_specs=pl.BlockSpec(memory_space=pltpu.VMEM),
        scratch_shapes=[
            pltpu.VMEM((N_DEV, H, SQ, DH), jnp.float32),
            pltpu.VMEM((N_DEV, H, SQ, DH), jnp.float32),
            pltpu.VMEM((N_DEV, 2, H, SQ), jnp.float32),
            pltpu.VMEM((SQ, D), jnp.float32),
            pltpu.SemaphoreType.DMA((N_DEV,)),
            pltpu.SemaphoreType.DMA((N_DEV,)),
            pltpu.SemaphoreType.DMA((N_DEV,)),
            pltpu.SemaphoreType.DMA((N_DEV,)),
            pltpu.SemaphoreType.DMA((N_DEV,)),
            pltpu.SemaphoreType.DMA((N_DEV,)),
            pltpu.SemaphoreType.DMA,
            pltpu.SemaphoreType.DMA,
        ],
        compiler_params=pltpu.CompilerParams(
            collective_id=0,
            vmem_limit_bytes=128 * 1024 * 1024,
        ),
    )(x, Wq, Wo, K_ext, V_ext)


# baseline (device time: 1612787 ns/iter reference)
import jax
import jax.numpy as jnp
from jax import lax
from jax.experimental import pallas as pl
from jax.experimental.pallas import tpu as pltpu

N_DEV = 8
SQ = 256
SKV = 4096
H = 8
DH = 128
D = 1024
KV_BLK = 256
N_BLK = SKV // KV_BLK
SCALE = 0.08838834764831843
NEG_BIG = -1e30


def _attn_body(q_ref, k_ref, v_ref, out_ref,
               q_buf, o_buf, ml_buf, k_stage, v_stage,
               kv_sems,
               q_send, q_recv, o_send, o_recv, ml_send, ml_recv,
               fin_send, fin_recv):
    me = lax.axis_index("i")
    right = lax.rem(me + 1, N_DEV)
    left = lax.rem(me + N_DEV - 1, N_DEV)

    barrier = pltpu.get_barrier_semaphore()
    pl.semaphore_signal(barrier, inc=1, device_id=(left,),
                        device_id_type=pl.DeviceIdType.MESH)
    pl.semaphore_signal(barrier, inc=1, device_id=(right,),
                        device_id_type=pl.DeviceIdType.MESH)
    pl.semaphore_wait(barrier, 2)

    def kv_copy(j, slot):
        k_cp = pltpu.make_async_copy(
            k_ref.at[0, pl.ds(j * KV_BLK, KV_BLK)],
            k_stage.at[slot], kv_sems.at[slot, 0])
        v_cp = pltpu.make_async_copy(
            v_ref.at[0, pl.ds(j * KV_BLK, KV_BLK)],
            v_stage.at[slot], kv_sems.at[slot, 1])
        return k_cp, v_cp

    for h in range(N_DEV):
        for cp in kv_copy(0, 0):
            cp.start()

        if h == 0:
            q_buf[0] = q_ref[...]
            o_buf[0] = jnp.zeros((H, SQ, DH), jnp.float32)
            ml_buf[0, 0] = jnp.full((H, SQ), NEG_BIG, jnp.float32)
            ml_buf[0, 1] = jnp.zeros((H, SQ), jnp.float32)

        def step(j, _, h=h):
            slot = lax.rem(j, 2)
            for cp in kv_copy(j, slot):
                cp.wait()

            @pl.when(j < N_BLK - 1)
            def _():
                for cp in kv_copy(j + 1, lax.rem(j + 1, 2)):
                    cp.start()

            q = q_buf[h]
            kj = k_stage[slot]
            vj = v_stage[slot]
            s = lax.dot_general(
                q, kj, (((2,), (2,)), ((0,), (1,))),
                preferred_element_type=jnp.float32) * SCALE
            m = ml_buf[h, 0]
            l = ml_buf[h, 1]
            mj = jnp.max(s, axis=-1)
            m_new = jnp.maximum(m, mj)
            alpha = jnp.exp(m - m_new)
            p = jnp.exp(s - m_new[:, :, None])
            ml_buf[h, 0] = m_new
            ml_buf[h, 1] = l * alpha + jnp.sum(p, axis=-1)
            pv = lax.dot_general(
                p, vj, (((2,), (0,)), ((0,), (1,))),
                preferred_element_type=jnp.float32)
            o_buf[h] = o_buf[h] * alpha[:, :, None] + pv
            return 0

        lax.fori_loop(0, N_BLK, step, 0)

        if h < N_DEV - 1:
            rq = pltpu.make_async_remote_copy(
                src_ref=q_buf.at[h], dst_ref=q_buf.at[h + 1],
                send_sem=q_send.at[h], recv_sem=q_recv.at[h + 1],
                device_id=(right,), device_id_type=pl.DeviceIdType.MESH)
            ro = pltpu.make_async_remote_copy(
                src_ref=o_buf.at[h], dst_ref=o_buf.at[h + 1],
                send_sem=o_send.at[h], recv_sem=o_recv.at[h + 1],
                device_id=(right,), device_id_type=pl.DeviceIdType.MESH)
            rml = pltpu.make_async_remote_copy(
                src_ref=ml_buf.at[h], dst_ref=ml_buf.at[h + 1],
                send_sem=ml_send.at[h], recv_sem=ml_recv.at[h + 1],
                device_id=(right,), device_id_type=pl.DeviceIdType.MESH)
            rq.start()
            ro.start()
            rml.start()
            rq.wait()
            ro.wait()
            rml.wait()
        else:
            l = ml_buf[h, 1]
            o_buf[h] = o_buf[h] / l[:, :, None]
            rfin = pltpu.make_async_remote_copy(
                src_ref=o_buf.at[h], dst_ref=out_ref,
                send_sem=fin_send, recv_sem=fin_recv,
                device_id=(right,), device_id_type=pl.DeviceIdType.MESH)
            rfin.start()
            rfin.wait()


def kernel(x, Wq, Wo, K_ext, V_ext):
    q = jnp.dot(x[0], Wq, preferred_element_type=jnp.float32)
    q = q.reshape(SQ, H, DH).transpose(1, 0, 2)

    o = pl.pallas_call(
        _attn_body,
        out_shape=jax.ShapeDtypeStruct((H, SQ, DH), jnp.float32),
        in_specs=[
            pl.BlockSpec(memory_space=pltpu.VMEM),
            pl.BlockSpec(memory_space=pl.ANY),
            pl.BlockSpec(memory_space=pl.ANY),
        ],
        out_specs=pl.BlockSpec(memory_space=pltpu.VMEM),
        scratch_shapes=[
            pltpu.VMEM((N_DEV, H, SQ, DH), jnp.float32),
            pltpu.VMEM((N_DEV, H, SQ, DH), jnp.float32),
            pltpu.VMEM((N_DEV, 2, H, SQ), jnp.float32),
            pltpu.VMEM((2, KV_BLK, H, DH), jnp.float32),
            pltpu.VMEM((2, KV_BLK, H, DH), jnp.float32),
            pltpu.SemaphoreType.DMA((2, 2)),
            pltpu.SemaphoreType.DMA((N_DEV,)),
            pltpu.SemaphoreType.DMA((N_DEV,)),
            pltpu.SemaphoreType.DMA((N_DEV,)),
            pltpu.SemaphoreType.DMA((N_DEV,)),
            pltpu.SemaphoreType.DMA((N_DEV,)),
            pltpu.SemaphoreType.DMA((N_DEV,)),
            pltpu.SemaphoreType.DMA,
            pltpu.SemaphoreType.DMA,
        ],
        compiler_params=pltpu.CompilerParams(
            collective_id=0,
            vmem_limit_bytes=60 * 1024 * 1024,
        ),
    )(q, K_ext, V_ext)

    res = o.transpose(1, 0, 2).reshape(SQ, H * DH)
    return jnp.dot(res, Wo, preferred_element_type=jnp.float32)[None]


# device time: 456028 ns/iter; 3.5366x vs baseline; 3.5366x over previous
import jax
import jax.numpy as jnp
from jax import lax
from jax.experimental import pallas as pl
from jax.experimental.pallas import tpu as pltpu

N_DEV = 8
SQ = 256
SKV = 4096
H = 8
DH = 128
D = 1024
KV_BLK = 256
N_BLK = SKV // KV_BLK
SCALE = 0.08838834764831843
NEG_BIG = -1e30


def _attn_body(q_ref, k_ref, v_ref, out_ref,
               q_buf, o_buf, ml_buf, k_stage, v_stage,
               kv_sems,
               q_send, q_recv, o_send, o_recv, ml_send, ml_recv,
               fin_send, fin_recv):
    me = lax.axis_index("i")
    right = lax.rem(me + 1, N_DEV)
    left = lax.rem(me + N_DEV - 1, N_DEV)

    barrier = pltpu.get_barrier_semaphore()
    pl.semaphore_signal(barrier, inc=1, device_id=(left,),
                        device_id_type=pl.DeviceIdType.MESH)
    pl.semaphore_signal(barrier, inc=1, device_id=(right,),
                        device_id_type=pl.DeviceIdType.MESH)
    pl.semaphore_wait(barrier, 2)

    def kv_copy(j, slot):
        k_cp = pltpu.make_async_copy(
            k_ref.at[:, pl.ds(j * KV_BLK, KV_BLK), :],
            k_stage.at[slot], kv_sems.at[slot, 0])
        v_cp = pltpu.make_async_copy(
            v_ref.at[:, pl.ds(j * KV_BLK, KV_BLK), :],
            v_stage.at[slot], kv_sems.at[slot, 1])
        return k_cp, v_cp

    for h in range(N_DEV):
        for cp in kv_copy(0, 0):
            cp.start()

        if h == 0:
            q_buf[0] = q_ref[...]
            o_buf[0] = jnp.zeros((H, SQ, DH), jnp.float32)
            ml_buf[0, 0] = jnp.full((H, SQ), NEG_BIG, jnp.float32)
            ml_buf[0, 1] = jnp.zeros((H, SQ), jnp.float32)

        def step(j, _, h=h):
            slot = lax.rem(j, 2)
            for cp in kv_copy(j, slot):
                cp.wait()

            @pl.when(j < N_BLK - 1)
            def _():
                for cp in kv_copy(j + 1, lax.rem(j + 1, 2)):
                    cp.start()

            q = q_buf[h]
            kj = k_stage[slot]
            vj = v_stage[slot]
            s = lax.dot_general(
                q, kj, (((2,), (2,)), ((0,), (0,))),
                preferred_element_type=jnp.float32) * SCALE
            m = ml_buf[h, 0]
            l = ml_buf[h, 1]
            mj = jnp.max(s, axis=-1)
            m_new = jnp.maximum(m, mj)
            alpha = jnp.exp(m - m_new)
            p = jnp.exp(s - m_new[:, :, None])
            ml_buf[h, 0] = m_new
            ml_buf[h, 1] = l * alpha + jnp.sum(p, axis=-1)
            pv = lax.dot_general(
                p, vj, (((2,), (1,)), ((0,), (0,))),
                preferred_element_type=jnp.float32)
            o_buf[h] = o_buf[h] * alpha[:, :, None] + pv
            return 0

        lax.fori_loop(0, N_BLK, step, 0)

        if h < N_DEV - 1:
            rq = pltpu.make_async_remote_copy(
                src_ref=q_buf.at[h], dst_ref=q_buf.at[h + 1],
                send_sem=q_send.at[h], recv_sem=q_recv.at[h + 1],
                device_id=(right,), device_id_type=pl.DeviceIdType.MESH)
            ro = pltpu.make_async_remote_copy(
                src_ref=o_buf.at[h], dst_ref=o_buf.at[h + 1],
                send_sem=o_send.at[h], recv_sem=o_recv.at[h + 1],
                device_id=(right,), device_id_type=pl.DeviceIdType.MESH)
            rml = pltpu.make_async_remote_copy(
                src_ref=ml_buf.at[h], dst_ref=ml_buf.at[h + 1],
                send_sem=ml_send.at[h], recv_sem=ml_recv.at[h + 1],
                device_id=(right,), device_id_type=pl.DeviceIdType.MESH)
            rq.start()
            ro.start()
            rml.start()
            rq.wait()
            ro.wait()
            rml.wait()
        else:
            l = ml_buf[h, 1]
            o_buf[h] = o_buf[h] / l[:, :, None]
            rfin = pltpu.make_async_remote_copy(
                src_ref=o_buf.at[h], dst_ref=out_ref,
                send_sem=fin_send, recv_sem=fin_recv,
                device_id=(right,), device_id_type=pl.DeviceIdType.MESH)
            rfin.start()
            rfin.wait()


def kernel(x, Wq, Wo, K_ext, V_ext):
    q = jnp.dot(x[0], Wq, preferred_element_type=jnp.float32)
    q = q.reshape(SQ, H, DH).transpose(1, 0, 2)
    k_hm = K_ext[0].transpose(1, 0, 2)
    v_hm = V_ext[0].transpose(1, 0, 2)

    o = pl.pallas_call(
        _attn_body,
        out_shape=jax.ShapeDtypeStruct((H, SQ, DH), jnp.float32),
        in_specs=[
            pl.BlockSpec(memory_space=pltpu.VMEM),
            pl.BlockSpec(memory_space=pl.ANY),
            pl.BlockSpec(memory_space=pl.ANY),
        ],
        out_specs=pl.BlockSpec(memory_space=pltpu.VMEM),
        scratch_shapes=[
            pltpu.VMEM((N_DEV, H, SQ, DH), jnp.float32),
            pltpu.VMEM((N_DEV, H, SQ, DH), jnp.float32),
            pltpu.VMEM((N_DEV, 2, H, SQ), jnp.float32),
            pltpu.VMEM((2, H, KV_BLK, DH), jnp.float32),
            pltpu.VMEM((2, H, KV_BLK, DH), jnp.float32),
            pltpu.SemaphoreType.DMA((2, 2)),
            pltpu.SemaphoreType.DMA((N_DEV,)),
            pltpu.SemaphoreType.DMA((N_DEV,)),
            pltpu.SemaphoreType.DMA((N_DEV,)),
            pltpu.SemaphoreType.DMA((N_DEV,)),
            pltpu.SemaphoreType.DMA((N_DEV,)),
            pltpu.SemaphoreType.DMA((N_DEV,)),
            pltpu.SemaphoreType.DMA,
            pltpu.SemaphoreType.DMA,
        ],
        compiler_params=pltpu.CompilerParams(
            collective_id=0,
            vmem_limit_bytes=60 * 1024 * 1024,
        ),
    )(q, k_hm, v_hm)

    res = o.transpose(1, 0, 2).reshape(SQ, H * DH)
    return jnp.dot(res, Wo, preferred_element_type=jnp.float32)[None]


# device time: 367745 ns/iter; 4.3856x vs baseline; 1.2401x over previous
import jax
import jax.numpy as jnp
from jax import lax
from jax.experimental import pallas as pl
from jax.experimental.pallas import tpu as pltpu

N_DEV = 8
SQ = 256
SKV = 4096
H = 8
DH = 128
D = 1024
KV_BLK = 256
N_BLK = SKV // KV_BLK
SCALE = 0.08838834764831843
NEG_BIG = -1e30


def _attn_body(q_ref, k_ref, v_ref, out_ref,
               q_buf, o_buf, ml_buf, k_stage, v_stage,
               kv_sems,
               q_send, q_recv, o_send, o_recv, ml_send, ml_recv,
               fin_send, fin_recv):
    me = lax.axis_index("i")
    right = lax.rem(me + 1, N_DEV)
    left = lax.rem(me + N_DEV - 1, N_DEV)

    barrier = pltpu.get_barrier_semaphore()
    pl.semaphore_signal(barrier, inc=1, device_id=(left,),
                        device_id_type=pl.DeviceIdType.MESH)
    pl.semaphore_signal(barrier, inc=1, device_id=(right,),
                        device_id_type=pl.DeviceIdType.MESH)
    pl.semaphore_wait(barrier, 2)

    def kv_copy(j, slot):
        k_cp = pltpu.make_async_copy(
            k_ref.at[:, pl.ds(j * KV_BLK, KV_BLK), :],
            k_stage.at[slot], kv_sems.at[slot, 0])
        v_cp = pltpu.make_async_copy(
            v_ref.at[:, pl.ds(j * KV_BLK, KV_BLK), :],
            v_stage.at[slot], kv_sems.at[slot, 1])
        return k_cp, v_cp

    def ring_rdma(buf, send_sems, recv_sems, h_src, h_dst):
        return pltpu.make_async_remote_copy(
            src_ref=buf.at[h_src], dst_ref=buf.at[h_dst],
            send_sem=send_sems.at[h_src], recv_sem=recv_sems.at[h_dst],
            device_id=(right,), device_id_type=pl.DeviceIdType.MESH)

    for h in range(N_DEV):
        for cp in kv_copy(0, 0):
            cp.start()

        if h == 0:
            q_buf[0] = q_ref[...]
            o_buf[0] = jnp.zeros((H, SQ, DH), jnp.float32)
            ml_buf[0, 0] = jnp.full((H, SQ), NEG_BIG, jnp.float32)
            ml_buf[0, 1] = jnp.zeros((H, SQ), jnp.float32)
        else:
            ring_rdma(q_buf, q_send, q_recv, h, h).wait_recv()
            ring_rdma(o_buf, o_send, o_recv, h, h).wait_recv()
            ring_rdma(ml_buf, ml_send, ml_recv, h, h).wait_recv()

        if h < N_DEV - 1:
            ring_rdma(q_buf, q_send, q_recv, h, h + 1).start()

        def step(j, _, h=h):
            slot = lax.rem(j, 2)
            for cp in kv_copy(j, slot):
                cp.wait()

            @pl.when(j < N_BLK - 1)
            def _():
                for cp in kv_copy(j + 1, lax.rem(j + 1, 2)):
                    cp.start()

            q = q_buf[h]
            kj = k_stage[slot]
            vj = v_stage[slot]
            s = lax.dot_general(
                q, kj, (((2,), (2,)), ((0,), (0,))),
                preferred_element_type=jnp.float32) * SCALE
            m = ml_buf[h, 0]
            l = ml_buf[h, 1]
            mj = jnp.max(s, axis=-1)
            m_new = jnp.maximum(m, mj)
            alpha = jnp.exp(m - m_new)
            p = jnp.exp(s - m_new[:, :, None])
            ml_buf[h, 0] = m_new
            ml_buf[h, 1] = l * alpha + jnp.sum(p, axis=-1)
            pv = lax.dot_general(
                p, vj, (((2,), (1,)), ((0,), (0,))),
                preferred_element_type=jnp.float32)
            o_buf[h] = o_buf[h] * alpha[:, :, None] + pv
            return 0

        lax.fori_loop(0, N_BLK, step, 0)

        if h < N_DEV - 1:
            ring_rdma(o_buf, o_send, o_recv, h, h + 1).start()
            ring_rdma(ml_buf, ml_send, ml_recv, h, h + 1).start()
        else:
            l = ml_buf[h, 1]
            o_buf[h] = o_buf[h] / l[:, :, None]
            rfin = pltpu.make_async_remote_copy(
                src_ref=o_buf.at[h], dst_ref=out_ref,
                send_sem=fin_send, recv_sem=fin_recv,
                device_id=(right,), device_id_type=pl.DeviceIdType.MESH)
            rfin.start()
            rfin.wait()

    for h in range(N_DEV - 1):
        ring_rdma(q_buf, q_send, q_recv, h, h + 1).wait_send()
        ring_rdma(o_buf, o_send, o_recv, h, h + 1).wait_send()
        ring_rdma(ml_buf, ml_send, ml_recv, h, h + 1).wait_send()


def kernel(x, Wq, Wo, K_ext, V_ext):
    q = jnp.dot(x[0], Wq, preferred_element_type=jnp.float32)
    q = q.reshape(SQ, H, DH).transpose(1, 0, 2)
    k_hm = K_ext[0].transpose(1, 0, 2)
    v_hm = V_ext[0].transpose(1, 0, 2)

    o = pl.pallas_call(
        _attn_body,
        out_shape=jax.ShapeDtypeStruct((H, SQ, DH), jnp.float32),
        in_specs=[
            pl.BlockSpec(memory_space=pltpu.VMEM),
            pl.BlockSpec(memory_space=pl.ANY),
            pl.BlockSpec(memory_space=pl.ANY),
        ],
        out_specs=pl.BlockSpec(memory_space=pltpu.VMEM),
        scratch_shapes=[
            pltpu.VMEM((N_DEV, H, SQ, DH), jnp.float32),
            pltpu.VMEM((N_DEV, H, SQ, DH), jnp.float32),
            pltpu.VMEM((N_DEV, 2, H, SQ), jnp.float32),
            pltpu.VMEM((2, H, KV_BLK, DH), jnp.float32),
            pltpu.VMEM((2, H, KV_BLK, DH), jnp.float32),
            pltpu.SemaphoreType.DMA((2, 2)),
            pltpu.SemaphoreType.DMA((N_DEV,)),
            pltpu.SemaphoreType.DMA((N_DEV,)),
            pltpu.SemaphoreType.DMA((N_DEV,)),
            pltpu.SemaphoreType.DMA((N_DEV,)),
            pltpu.SemaphoreType.DMA((N_DEV,)),
            pltpu.SemaphoreType.DMA((N_DEV,)),
            pltpu.SemaphoreType.DMA,
            pltpu.SemaphoreType.DMA,
        ],
        compiler_params=pltpu.CompilerParams(
            collective_id=0,
            vmem_limit_bytes=60 * 1024 * 1024,
        ),
    )(q, k_hm, v_hm)

    res = o.transpose(1, 0, 2).reshape(SQ, H * DH)
    return jnp.dot(res, Wo, preferred_element_type=jnp.float32)[None]


# device time: 291797 ns/iter; 5.5271x vs baseline; 1.2603x over previous
import jax
import jax.numpy as jnp
from jax import lax
from jax.experimental import pallas as pl
from jax.experimental.pallas import tpu as pltpu

N_DEV = 8
SQ = 256
SKV = 4096
H = 8
DH = 128
D = 1024
KV_BLK = 512
N_BLK = SKV // KV_BLK
SCALE = 0.08838834764831843
NEG_BIG = -1e30


def _attn_body(q_ref, k_ref, v_ref, out_ref,
               q_buf, o_buf, ml_buf, k_stage, v_stage,
               kv_sems,
               q_send, q_recv, o_send, o_recv, ml_send, ml_recv,
               fin_send, fin_recv):
    me = lax.axis_index("i")
    right = lax.rem(me + 1, N_DEV)
    left = lax.rem(me + N_DEV - 1, N_DEV)

    barrier = pltpu.get_barrier_semaphore()
    pl.semaphore_signal(barrier, inc=1, device_id=(left,),
                        device_id_type=pl.DeviceIdType.MESH)
    pl.semaphore_signal(barrier, inc=1, device_id=(right,),
                        device_id_type=pl.DeviceIdType.MESH)
    pl.semaphore_wait(barrier, 2)

    def kv_copy(j, slot):
        k_cp = pltpu.make_async_copy(
            k_ref.at[:, pl.ds(j * KV_BLK, KV_BLK), :],
            k_stage.at[slot], kv_sems.at[slot, 0])
        v_cp = pltpu.make_async_copy(
            v_ref.at[:, pl.ds(j * KV_BLK, KV_BLK), :],
            v_stage.at[slot], kv_sems.at[slot, 1])
        return k_cp, v_cp

    def ring_rdma(buf, send_sems, recv_sems, h_src, h_dst):
        return pltpu.make_async_remote_copy(
            src_ref=buf.at[h_src], dst_ref=buf.at[h_dst],
            send_sem=send_sems.at[h_src], recv_sem=recv_sems.at[h_dst],
            device_id=(right,), device_id_type=pl.DeviceIdType.MESH)

    for h in range(N_DEV):
        for cp in kv_copy(0, 0):
            cp.start()

        if h == 0:
            q_buf[0] = q_ref[...]
            o_buf[0] = jnp.zeros((H, SQ, DH), jnp.float32)
            ml_buf[0, 0] = jnp.full((H, SQ), NEG_BIG, jnp.float32)
            ml_buf[0, 1] = jnp.zeros((H, SQ), jnp.float32)
        else:
            ring_rdma(q_buf, q_send, q_recv, h, h).wait_recv()
            ring_rdma(o_buf, o_send, o_recv, h, h).wait_recv()
            ring_rdma(ml_buf, ml_send, ml_recv, h, h).wait_recv()

        if h < N_DEV - 1:
            ring_rdma(q_buf, q_send, q_recv, h, h + 1).start()

        def step(j, _, h=h):
            slot = lax.rem(j, 2)
            for cp in kv_copy(j, slot):
                cp.wait()

            @pl.when(j < N_BLK - 1)
            def _():
                for cp in kv_copy(j + 1, lax.rem(j + 1, 2)):
                    cp.start()

            q = q_buf[h]
            kj = k_stage[slot]
            vj = v_stage[slot]
            s = lax.dot_general(
                q, kj, (((2,), (2,)), ((0,), (0,))),
                preferred_element_type=jnp.float32)
            m = ml_buf[h, 0]
            l = ml_buf[h, 1]
            mj = jnp.max(s, axis=-1)
            m_new = jnp.maximum(m, mj)
            alpha = jnp.exp(m - m_new)
            p = jnp.exp(s - m_new[:, :, None])
            ml_buf[h, 0] = m_new
            ml_buf[h, 1] = l * alpha + jnp.sum(p, axis=-1)
            pv = lax.dot_general(
                p.astype(jnp.bfloat16), vj, (((2,), (1,)), ((0,), (0,))),
                preferred_element_type=jnp.float32)
            o_buf[h] = o_buf[h] * alpha[:, :, None] + pv
            return 0

        lax.fori_loop(0, N_BLK, step, 0)

        if h < N_DEV - 1:
            ring_rdma(o_buf, o_send, o_recv, h, h + 1).start()
            ring_rdma(ml_buf, ml_send, ml_recv, h, h + 1).start()
        else:
            l = ml_buf[h, 1]
            o_buf[h] = o_buf[h] / l[:, :, None]
            rfin = pltpu.make_async_remote_copy(
                src_ref=o_buf.at[h], dst_ref=out_ref,
                send_sem=fin_send, recv_sem=fin_recv,
                device_id=(right,), device_id_type=pl.DeviceIdType.MESH)
            rfin.start()
            rfin.wait()

    for h in range(N_DEV - 1):
        ring_rdma(q_buf, q_send, q_recv, h, h + 1).wait_send()
        ring_rdma(o_buf, o_send, o_recv, h, h + 1).wait_send()
        ring_rdma(ml_buf, ml_send, ml_recv, h, h + 1).wait_send()


def kernel(x, Wq, Wo, K_ext, V_ext):
    q = jnp.dot(x[0], Wq, preferred_element_type=jnp.float32)
    q = (q * SCALE).reshape(SQ, H, DH).transpose(1, 0, 2)
    q = q.astype(jnp.bfloat16)
    k_hm = K_ext[0].transpose(1, 0, 2).astype(jnp.bfloat16)
    v_hm = V_ext[0].transpose(1, 0, 2).astype(jnp.bfloat16)

    o = pl.pallas_call(
        _attn_body,
        out_shape=jax.ShapeDtypeStruct((H, SQ, DH), jnp.float32),
        in_specs=[
            pl.BlockSpec(memory_space=pltpu.VMEM),
            pl.BlockSpec(memory_space=pl.ANY),
            pl.BlockSpec(memory_space=pl.ANY),
        ],
        out_specs=pl.BlockSpec(memory_space=pltpu.VMEM),
        scratch_shapes=[
            pltpu.VMEM((N_DEV, H, SQ, DH), jnp.bfloat16),
            pltpu.VMEM((N_DEV, H, SQ, DH), jnp.float32),
            pltpu.VMEM((N_DEV, 2, H, SQ), jnp.float32),
            pltpu.VMEM((2, H, KV_BLK, DH), jnp.bfloat16),
            pltpu.VMEM((2, H, KV_BLK, DH), jnp.bfloat16),
            pltpu.SemaphoreType.DMA((2, 2)),
            pltpu.SemaphoreType.DMA((N_DEV,)),
            pltpu.SemaphoreType.DMA((N_DEV,)),
            pltpu.SemaphoreType.DMA((N_DEV,)),
            pltpu.SemaphoreType.DMA((N_DEV,)),
            pltpu.SemaphoreType.DMA((N_DEV,)),
            pltpu.SemaphoreType.DMA((N_DEV,)),
            pltpu.SemaphoreType.DMA,
            pltpu.SemaphoreType.DMA,
        ],
        compiler_params=pltpu.CompilerParams(
            collective_id=0,
            vmem_limit_bytes=60 * 1024 * 1024,
        ),
    )(q, k_hm, v_hm)

    res = o.transpose(1, 0, 2).reshape(SQ, H * DH)
    return jnp.dot(res, Wo, preferred_element_type=jnp.float32)[None]


# device time: 207912 ns/iter; 7.7571x vs baseline; 1.4035x over previous
import jax
import jax.numpy as jnp
from jax import lax
from jax.experimental import pallas as pl
from jax.experimental.pallas import tpu as pltpu

N_DEV = 8
SQ = 256
SKV = 4096
H = 8
DH = 128
D = 1024
KV_BLK = 512
N_BLK = SKV // KV_BLK
SCALE = 0.08838834764831843
NEG_BIG = -1e30


def _attn_body(q_ref, k_ref, v_ref, out_ref,
               q_buf, o_buf, ml_buf, o_loc, ml_loc,
               q_send, q_recv, o_send, o_recv, ml_send, ml_recv,
               fin_send, fin_recv):
    me = lax.axis_index("i")
    right = lax.rem(me + 1, N_DEV)
    left = lax.rem(me + N_DEV - 1, N_DEV)

    barrier = pltpu.get_barrier_semaphore()
    pl.semaphore_signal(barrier, inc=1, device_id=(left,),
                        device_id_type=pl.DeviceIdType.MESH)
    pl.semaphore_signal(barrier, inc=1, device_id=(right,),
                        device_id_type=pl.DeviceIdType.MESH)
    pl.semaphore_wait(barrier, 2)

    def ring_rdma(buf, send_sems, recv_sems, h_src, h_dst):
        return pltpu.make_async_remote_copy(
            src_ref=buf.at[h_src], dst_ref=buf.at[h_dst],
            send_sem=send_sems.at[h_src], recv_sem=recv_sems.at[h_dst],
            device_id=(right,), device_id_type=pl.DeviceIdType.MESH)

    q_buf[0] = q_ref[...]

    for h in range(N_DEV):
        if h > 0:
            ring_rdma(q_buf, q_send, q_recv, h, h).wait_recv()
        if h < N_DEV - 1:
            ring_rdma(q_buf, q_send, q_recv, h, h + 1).start()

        o_loc[...] = jnp.zeros((H, SQ, DH), jnp.float32)
        ml_loc[0] = jnp.full((H, SQ), NEG_BIG, jnp.float32)
        ml_loc[1] = jnp.zeros((H, SQ), jnp.float32)

        def step(j, _, h=h):
            q = q_buf[h]
            kj = k_ref[:, pl.ds(j * KV_BLK, KV_BLK), :]
            vj = v_ref[:, pl.ds(j * KV_BLK, KV_BLK), :]
            s = lax.dot_general(
                q, kj, (((2,), (2,)), ((0,), (0,))),
                preferred_element_type=jnp.float32)
            m = ml_loc[0]
            l = ml_loc[1]
            mj = jnp.max(s, axis=-1)
            m_new = jnp.maximum(m, mj)
            alpha = jnp.exp(m - m_new)
            p = jnp.exp(s - m_new[:, :, None])
            ml_loc[0] = m_new
            ml_loc[1] = l * alpha + jnp.sum(p, axis=-1)
            pv = lax.dot_general(
                p.astype(jnp.bfloat16), vj, (((2,), (1,)), ((0,), (0,))),
                preferred_element_type=jnp.float32)
            o_loc[...] = o_loc[...] * alpha[:, :, None] + pv
            return 0

        lax.fori_loop(0, N_BLK, step, 0)

        if h == 0:
            o_buf[0] = o_loc[...]
            ml_buf[0, 0] = ml_loc[0]
            ml_buf[0, 1] = ml_loc[1]
        else:
            ring_rdma(o_buf, o_send, o_recv, h, h).wait_recv()
            ring_rdma(ml_buf, ml_send, ml_recv, h, h).wait_recv()
            m_in = ml_buf[h, 0]
            l_in = ml_buf[h, 1]
            m_loc = ml_loc[0]
            l_loc = ml_loc[1]
            m_new = jnp.maximum(m_in, m_loc)
            a_in = jnp.exp(m_in - m_new)
            a_loc = jnp.exp(m_loc - m_new)
            ml_buf[h, 0] = m_new
            ml_buf[h, 1] = l_in * a_in + l_loc * a_loc
            o_buf[h] = (o_buf[h] * a_in[:, :, None]
                        + o_loc[...] * a_loc[:, :, None])

        if h < N_DEV - 1:
            ring_rdma(o_buf, o_send, o_recv, h, h + 1).start()
            ring_rdma(ml_buf, ml_send, ml_recv, h, h + 1).start()
        else:
            l = ml_buf[h, 1]
            o_buf[h] = o_buf[h] / l[:, :, None]
            rfin = pltpu.make_async_remote_copy(
                src_ref=o_buf.at[h], dst_ref=out_ref,
                send_sem=fin_send, recv_sem=fin_recv,
                device_id=(right,), device_id_type=pl.DeviceIdType.MESH)
            rfin.start()
            rfin.wait()

    for h in range(N_DEV - 1):
        ring_rdma(q_buf, q_send, q_recv, h, h + 1).wait_send()
        ring_rdma(o_buf, o_send, o_recv, h, h + 1).wait_send()
        ring_rdma(ml_buf, ml_send, ml_recv, h, h + 1).wait_send()


def kernel(x, Wq, Wo, K_ext, V_ext):
    q = jnp.dot(x[0], Wq, preferred_element_type=jnp.float32)
    q = (q * SCALE).reshape(SQ, H, DH).transpose(1, 0, 2)
    q = q.astype(jnp.bfloat16)
    k_hm = K_ext[0].transpose(1, 0, 2).astype(jnp.bfloat16)
    v_hm = V_ext[0].transpose(1, 0, 2).astype(jnp.bfloat16)

    o = pl.pallas_call(
        _attn_body,
        out_shape=jax.ShapeDtypeStruct((H, SQ, DH), jnp.float32),
        in_specs=[pl.BlockSpec(memory_space=pltpu.VMEM)] * 3,
        out_specs=pl.BlockSpec(memory_space=pltpu.VMEM),
        scratch_shapes=[
            pltpu.VMEM((N_DEV, H, SQ, DH), jnp.bfloat16),
            pltpu.VMEM((N_DEV, H, SQ, DH), jnp.float32),
            pltpu.VMEM((N_DEV, 2, H, SQ), jnp.float32),
            pltpu.VMEM((H, SQ, DH), jnp.float32),
            pltpu.VMEM((2, H, SQ), jnp.float32),
            pltpu.SemaphoreType.DMA((N_DEV,)),
            pltpu.SemaphoreType.DMA((N_DEV,)),
            pltpu.SemaphoreType.DMA((N_DEV,)),
            pltpu.SemaphoreType.DMA((N_DEV,)),
            pltpu.SemaphoreType.DMA((N_DEV,)),
            pltpu.SemaphoreType.DMA((N_DEV,)),
            pltpu.SemaphoreType.DMA,
            pltpu.SemaphoreType.DMA,
        ],
        compiler_params=pltpu.CompilerParams(
            collective_id=0,
            vmem_limit_bytes=60 * 1024 * 1024,
        ),
    )(q, k_hm, v_hm)

    res = o.transpose(1, 0, 2).reshape(SQ, H * DH)
    return jnp.dot(res, Wo, preferred_element_type=jnp.float32)[None]


# device time: 204390 ns/iter; 7.8907x vs baseline; 1.0172x over previous
import jax
import jax.numpy as jnp
from jax import lax
from jax.experimental import pallas as pl
from jax.experimental.pallas import tpu as pltpu

N_DEV = 8
SQ = 256
SKV = 4096
H = 8
DH = 128
D = 1024
KV_BLK = 512
N_BLK = SKV // KV_BLK
SCALE = 0.08838834764831843
NEG_BIG = -1e30


def _attn_body(q_ref, k_ref, v_ref, out_ref,
               q_buf, o_buf, ml_buf, o_loc, ml_loc,
               q_send, q_recv, o_send, o_recv, ml_send, ml_recv,
               fin_send, fin_recv):
    me = lax.axis_index("i")
    right = lax.rem(me + 1, N_DEV)
    left = lax.rem(me + N_DEV - 1, N_DEV)

    barrier = pltpu.get_barrier_semaphore()
    pl.semaphore_signal(barrier, inc=1, device_id=(left,),
                        device_id_type=pl.DeviceIdType.MESH)
    pl.semaphore_signal(barrier, inc=1, device_id=(right,),
                        device_id_type=pl.DeviceIdType.MESH)
    pl.semaphore_wait(barrier, 2)

    def ring_rdma(buf, send_sems, recv_sems, h_src, h_dst):
        return pltpu.make_async_remote_copy(
            src_ref=buf.at[h_src], dst_ref=buf.at[h_dst],
            send_sem=send_sems.at[h_src], recv_sem=recv_sems.at[h_dst],
            device_id=(right,), device_id_type=pl.DeviceIdType.MESH)

    q_buf[0] = q_ref[...]

    for h in range(N_DEV):
        if h > 0:
            ring_rdma(q_buf, q_send, q_recv, h, h).wait_recv()
        if h < N_DEV - 1:
            ring_rdma(q_buf, q_send, q_recv, h, h + 1).start()

        q = q_buf[h]
        k0 = k_ref[:, :KV_BLK, :]
        v0 = v_ref[:, :KV_BLK, :]
        s0 = lax.dot_general(
            q, k0, (((2,), (2,)), ((0,), (0,))),
            preferred_element_type=jnp.float32)
        m0 = jnp.max(s0, axis=-1)
        p0 = jnp.exp(s0 - m0[:, :, None])
        ml_loc[0] = m0
        ml_loc[1] = jnp.sum(p0, axis=-1)
        o_loc[...] = lax.dot_general(
            p0.astype(jnp.bfloat16), v0, (((2,), (1,)), ((0,), (0,))),
            preferred_element_type=jnp.float32)

        def step(j, _, h=h):
            q = q_buf[h]
            kj = k_ref[:, pl.ds(j * KV_BLK, KV_BLK), :]
            vj = v_ref[:, pl.ds(j * KV_BLK, KV_BLK), :]
            s = lax.dot_general(
                q, kj, (((2,), (2,)), ((0,), (0,))),
                preferred_element_type=jnp.float32)
            m = ml_loc[0]
            l = ml_loc[1]
            mj = jnp.max(s, axis=-1)
            m_new = jnp.maximum(m, mj)
            alpha = jnp.exp(m - m_new)
            p = jnp.exp(s - m_new[:, :, None])
            ml_loc[0] = m_new
            ml_loc[1] = l * alpha + jnp.sum(p, axis=-1)
            pv = lax.dot_general(
                p.astype(jnp.bfloat16), vj, (((2,), (1,)), ((0,), (0,))),
                preferred_element_type=jnp.float32)
            o_loc[...] = o_loc[...] * alpha[:, :, None] + pv
            return 0

        lax.fori_loop(1, N_BLK, step, 0)

        if h == 0:
            o_buf[0] = o_loc[...]
            ml_buf[0, 0] = ml_loc[0]
            ml_buf[0, 1] = ml_loc[1]
        else:
            ring_rdma(o_buf, o_send, o_recv, h, h).wait_recv()
            ring_rdma(ml_buf, ml_send, ml_recv, h, h).wait_recv()
            m_in = ml_buf[h, 0]
            l_in = ml_buf[h, 1]
            m_loc = ml_loc[0]
            l_loc = ml_loc[1]
            m_new = jnp.maximum(m_in, m_loc)
            a_in = jnp.exp(m_in - m_new)
            a_loc = jnp.exp(m_loc - m_new)
            ml_buf[h, 0] = m_new
            ml_buf[h, 1] = l_in * a_in + l_loc * a_loc
            o_buf[h] = (o_buf[h] * a_in[:, :, None]
                        + o_loc[...] * a_loc[:, :, None])

        if h < N_DEV - 1:
            ring_rdma(o_buf, o_send, o_recv, h, h + 1).start()
            ring_rdma(ml_buf, ml_send, ml_recv, h, h + 1).start()
        else:
            l = ml_buf[h, 1]
            o_buf[h] = o_buf[h] / l[:, :, None]
            rfin = pltpu.make_async_remote_copy(
                src_ref=o_buf.at[h], dst_ref=out_ref,
                send_sem=fin_send, recv_sem=fin_recv,
                device_id=(right,), device_id_type=pl.DeviceIdType.MESH)
            rfin.start()
            rfin.wait()

    for h in range(N_DEV - 1):
        ring_rdma(q_buf, q_send, q_recv, h, h + 1).wait_send()
        ring_rdma(o_buf, o_send, o_recv, h, h + 1).wait_send()
        ring_rdma(ml_buf, ml_send, ml_recv, h, h + 1).wait_send()


def kernel(x, Wq, Wo, K_ext, V_ext):
    q = jnp.dot(x[0], Wq, preferred_element_type=jnp.float32)
    q = (q * SCALE).reshape(SQ, H, DH).transpose(1, 0, 2)
    q = q.astype(jnp.bfloat16)
    k_hm = K_ext[0].transpose(1, 0, 2).astype(jnp.bfloat16)
    v_hm = V_ext[0].transpose(1, 0, 2).astype(jnp.bfloat16)

    o = pl.pallas_call(
        _attn_body,
        out_shape=jax.ShapeDtypeStruct((H, SQ, DH), jnp.float32),
        in_specs=[pl.BlockSpec(memory_space=pltpu.VMEM)] * 3,
        out_specs=pl.BlockSpec(memory_space=pltpu.VMEM),
        scratch_shapes=[
            pltpu.VMEM((N_DEV, H, SQ, DH), jnp.bfloat16),
            pltpu.VMEM((N_DEV, H, SQ, DH), jnp.float32),
            pltpu.VMEM((N_DEV, 2, H, SQ), jnp.float32),
            pltpu.VMEM((H, SQ, DH), jnp.float32),
            pltpu.VMEM((2, H, SQ), jnp.float32),
            pltpu.SemaphoreType.DMA((N_DEV,)),
            pltpu.SemaphoreType.DMA((N_DEV,)),
            pltpu.SemaphoreType.DMA((N_DEV,)),
            pltpu.SemaphoreType.DMA((N_DEV,)),
            pltpu.SemaphoreType.DMA((N_DEV,)),
            pltpu.SemaphoreType.DMA((N_DEV,)),
            pltpu.SemaphoreType.DMA,
            pltpu.SemaphoreType.DMA,
        ],
        compiler_params=pltpu.CompilerParams(
            collective_id=0,
            vmem_limit_bytes=60 * 1024 * 1024,
        ),
    )(q, k_hm, v_hm)

    res = o.transpose(1, 0, 2).reshape(SQ, H * DH)
    return jnp.dot(res, Wo, preferred_element_type=jnp.float32)[None]
